# baseline (speedup 1.0000x reference)
import jax
import jax.numpy as jnp
from jax.experimental import pallas as pl


def _body(col_ref, row_ref, out_ref):
    B, C, HW = out_ref.shape
    D = C // 2
    H = W = 32
    colT = col_ref[:W, :].T
    rowT = row_ref[:H, :].T
    x_part = jnp.broadcast_to(colT[:, None, :], (D, H, W)).reshape(D, HW)
    y_part = jnp.broadcast_to(rowT[:, :, None], (D, H, W)).reshape(D, HW)
    pos = jnp.concatenate([x_part, y_part], axis=0)
    out_ref[...] = jnp.broadcast_to(pos[None], (B, C, HW))


def kernel(input_, row_embed, col_embed):
    B, _, H, W = input_.shape
    D = row_embed.shape[1]
    out = pl.pallas_call(
        _body,
        out_shape=jax.ShapeDtypeStruct((B, 2 * D, H * W), jnp.float32),
    )(col_embed, row_embed)
    return out.reshape(B, 2 * D, H, W)
